# baseline (speedup 1.0000x reference)
import jax
import jax.numpy as jnp
from jax.experimental import pallas as pl


def _add_kernel(x_ref, t_ref, o_ref):
    o_ref[...] = x_ref[...] + t_ref[...][None, :, :]


def kernel(x, table):
    batch, seq, d = x.shape
    bs = 128
    grid = (seq // bs,)
    return pl.pallas_call(
        _add_kernel,
        grid=grid,
        in_specs=[
            pl.BlockSpec((batch, bs, d), lambda i: (0, i, 0)),
            pl.BlockSpec((bs, d), lambda i: (i, 0)),
        ],
        out_specs=pl.BlockSpec((batch, bs, d), lambda i: (0, i, 0)),
        out_shape=jax.ShapeDtypeStruct((batch, seq, d), x.dtype),
    )(x, table)


# speedup vs baseline: 1.0897x; 1.0897x over previous
import jax
import jax.numpy as jnp
from jax.experimental import pallas as pl
from jax.experimental.pallas import tpu as pltpu


def _add_kernel(x_ref, t_ref, o_ref):
    o_ref[...] = x_ref[...] + t_ref[...][None, :, :]


def kernel(x, table):
    batch, seq, d = x.shape
    bs = 512
    grid = (seq // bs,)
    return pl.pallas_call(
        _add_kernel,
        grid=grid,
        compiler_params=pltpu.CompilerParams(
            dimension_semantics=("parallel",),
        ),
        in_specs=[
            pl.BlockSpec((batch, bs, d), lambda i: (0, i, 0)),
            pl.BlockSpec((bs, d), lambda i: (i, 0)),
        ],
        out_specs=pl.BlockSpec((batch, bs, d), lambda i: (0, i, 0)),
        out_shape=jax.ShapeDtypeStruct((batch, seq, d), x.dtype),
    )(x, table)


# speedup vs baseline: 1.1516x; 1.0568x over previous
import jax
import jax.numpy as jnp
from jax.experimental import pallas as pl
from jax.experimental.pallas import tpu as pltpu


def _add_kernel(x_ref, t_ref, o_ref):
    i = pl.program_id(0)
    bs = x_ref.shape[1]
    o_ref[...] = x_ref[...] + t_ref[pl.ds(i * bs, bs), :][None, :, :]


def kernel(x, table):
    batch, seq, d = x.shape
    bs = 512
    grid = (seq // bs,)
    return pl.pallas_call(
        _add_kernel,
        grid=grid,
        compiler_params=pltpu.CompilerParams(
            dimension_semantics=("arbitrary",),
        ),
        in_specs=[
            pl.BlockSpec((batch, bs, d), lambda i: (0, i, 0)),
            pl.BlockSpec((seq, d), lambda i: (0, 0)),
        ],
        out_specs=pl.BlockSpec((batch, bs, d), lambda i: (0, i, 0)),
        out_shape=jax.ShapeDtypeStruct((batch, seq, d), x.dtype),
    )(x, table)
